# Initial kernel scaffold; baseline (speedup 1.0000x reference)
import jax
import jax.numpy as jnp
from jax.experimental import pallas as pl


def kernel(x, gate_tensor, W_gate, W_up, W_down):
    raise NotImplementedError("write your pallas kernel here")


# speedup vs baseline: 1.6610x; 1.6610x over previous
import functools

import jax
import jax.numpy as jnp
from jax.experimental import pallas as pl
from jax.experimental.pallas import tpu as pltpu

T, D, F, E, TOP_K = 2048, 1024, 512, 8, 2


def _first_max_onehot(p):
    m = jnp.max(p, axis=-1, keepdims=True)
    eq = p == m
    lane = jax.lax.broadcasted_iota(jnp.int32, p.shape, 1)
    key = jnp.where(eq, lane, E)
    first = jnp.min(key, axis=-1, keepdims=True)
    return lane == first


def _dense_router_weights(x_f32, gate):
    logits = jnp.dot(x_f32, gate, preferred_element_type=jnp.float32)
    probs = jax.nn.softmax(logits, axis=-1)
    oh1 = _first_max_onehot(probs)
    p1 = jnp.max(probs, axis=-1, keepdims=True)
    probs2 = jnp.where(oh1, -jnp.inf, probs)
    oh2 = _first_max_onehot(probs2)
    p2 = jnp.max(probs2, axis=-1, keepdims=True)
    denom = p1 + p2 + 1e-20
    return (jnp.where(oh1, probs, 0.0) + jnp.where(oh2, probs, 0.0)) / denom


def _moe_kernel(x_f32_ref, xb_ref, gate_ref, wg_ref, wu_ref, wd_ref,
                out_ref, w_scr):
    e = pl.program_id(0)

    @pl.when(e == 0)
    def _():
        w_scr[...] = _dense_router_weights(x_f32_ref[...], gate_ref[...])
        out_ref[...] = jnp.zeros_like(out_ref)

    xb = xb_ref[...]
    hg = jnp.dot(xb, wg_ref[0], preferred_element_type=jnp.float32)
    hu = jnp.dot(xb, wu_ref[0], preferred_element_type=jnp.float32)
    h = (hg * jax.nn.sigmoid(hg) * hu).astype(jnp.bfloat16)
    y = jnp.dot(h, wd_ref[0], preferred_element_type=jnp.float32)
    lane = jax.lax.broadcasted_iota(jnp.int32, (1, E), 1)
    wcol = jnp.sum(jnp.where(lane == e, w_scr[...], 0.0), axis=-1,
                   keepdims=True)
    out_ref[...] += wcol * y


@jax.jit
def kernel(x, gate_tensor, W_gate, W_up, W_down):
    xb = x.astype(jnp.bfloat16)
    wg = W_gate.astype(jnp.bfloat16)
    wu = W_up.astype(jnp.bfloat16)
    wd = W_down.astype(jnp.bfloat16)
    return pl.pallas_call(
        _moe_kernel,
        grid=(E,),
        in_specs=[
            pl.BlockSpec((T, D), lambda e: (0, 0)),
            pl.BlockSpec((T, D), lambda e: (0, 0)),
            pl.BlockSpec((D, E), lambda e: (0, 0)),
            pl.BlockSpec((1, D, F), lambda e: (e, 0, 0)),
            pl.BlockSpec((1, D, F), lambda e: (e, 0, 0)),
            pl.BlockSpec((1, F, D), lambda e: (e, 0, 0)),
        ],
        out_specs=pl.BlockSpec((T, D), lambda e: (0, 0)),
        out_shape=jax.ShapeDtypeStruct((T, D), jnp.float32),
        scratch_shapes=[pltpu.VMEM((T, E), jnp.float32)],
    )(x, xb, gate_tensor, wg, wu, wd)
